# baseline (speedup 1.0000x reference)
import jax
import jax.numpy as jnp
from jax import lax
from jax.experimental import pallas as pl

NUM_OUTPUTS = 1000
ROWS = 4096
COLS = 26
FLAT = ROWS * COLS
BLOCK_R = 1024


def _onehot_block(idx_ref, out_ref):
    idx = idx_ref[...]
    iota = lax.broadcasted_iota(jnp.int32, (BLOCK_R, NUM_OUTPUTS), 1)
    out_ref[...] = (iota == idx).astype(jnp.int32)


def kernel(inputs):
    idx = inputs.reshape(FLAT, 1)
    grid = (FLAT // BLOCK_R,)
    out = pl.pallas_call(
        _onehot_block,
        grid=grid,
        in_specs=[pl.BlockSpec((BLOCK_R, 1), lambda i: (i, 0))],
        out_specs=pl.BlockSpec((BLOCK_R, NUM_OUTPUTS), lambda i: (i, 0)),
        out_shape=jax.ShapeDtypeStruct((FLAT, NUM_OUTPUTS), jnp.int32),
    )(idx)
    return out.reshape(ROWS, COLS, NUM_OUTPUTS)


# speedup vs baseline: 1.4722x; 1.4722x over previous
import jax
import jax.numpy as jnp
from jax import lax
from jax.experimental import pallas as pl

NUM_OUTPUTS = 1000
ROWS = 4096
COLS = 26
BLOCK_R = 64


def _onehot_block(idx_ref, out_ref):
    idx = idx_ref[...]
    iota = lax.broadcasted_iota(jnp.int32, (BLOCK_R, COLS, NUM_OUTPUTS), 2)
    out_ref[...] = (iota == idx[:, :, None]).astype(jnp.int32)


def kernel(inputs):
    grid = (ROWS // BLOCK_R,)
    out = pl.pallas_call(
        _onehot_block,
        grid=grid,
        in_specs=[pl.BlockSpec((BLOCK_R, COLS), lambda i: (i, 0))],
        out_specs=pl.BlockSpec((BLOCK_R, COLS, NUM_OUTPUTS), lambda i: (i, 0, 0)),
        out_shape=jax.ShapeDtypeStruct((ROWS, COLS, NUM_OUTPUTS), jnp.int32),
    )(inputs)
    return out
